# baseline (device time: 44309 ns/iter reference)
import os

import jax
import jax.numpy as jnp
from jax import lax
from jax.experimental import pallas as pl
from jax.experimental.pallas import tpu as pltpu

_VARIANT = os.environ.get("KERNEL_VARIANT", "full")

N_B = 2
S_PER = 512
N_H = 8
D = 64
HD = N_H * D
SCALE = D ** -0.5

N_CHUNKS = 4
CS = S_PER // 2


def kernel(Q, K, V):
    b, s_per, h, d = Q.shape
    Kf = K.reshape(b, s_per, h * d)
    Vf = V.reshape(b, s_per, h * d)

    def body(q_ref, k_ref, v_ref, out_ref, kv, dsend, drecv, fsend, frecv):
        my_x = lax.axis_index("x")
        my_y = lax.axis_index("y")
        x_partner = (1 - my_x, my_y)
        y_partner = (my_x, 1 - my_y)

        barrier = pltpu.get_barrier_semaphore()
        for nbr in (x_partner, y_partner):
            pl.semaphore_signal(
                barrier, inc=1, device_id=nbr,
                device_id_type=pl.DeviceIdType.MESH,
            )
        pl.semaphore_wait(barrier, 2)

        for bb in range(N_B):
            kv[0, bb, 0] = k_ref[bb].astype(jnp.bfloat16)
            kv[0, bb, 1] = v_ref[bb].astype(jnp.bfloat16)

        def chunk_idx(c):
            return c // 2, (c % 2) * CS

        direct = []
        for c in range(N_CHUNKS):
            t, s0 = chunk_idx(c)
            direct.append(pltpu.make_async_remote_copy(
                src_ref=kv.at[0, my_y, t, pl.ds(s0, CS)],
                dst_ref=kv.at[1, my_y, t, pl.ds(s0, CS)],
                send_sem=dsend.at[c], recv_sem=drecv.at[c],
                device_id=x_partner, device_id_type=pl.DeviceIdType.MESH,
            ))
        if _VARIANT != "compute":
            for r in direct:
                r.start()

        fwd = []
        for c in range(N_CHUNKS):
            t, s0 = chunk_idx(c)
            fwd.append(pltpu.make_async_remote_copy(
                src_ref=kv.at[1, my_y, t, pl.ds(s0, CS)],
                dst_ref=kv.at[1, my_y, t, pl.ds(s0, CS)],
                send_sem=fsend.at[c], recv_sem=frecv.at[c],
                device_id=y_partner, device_id_type=pl.DeviceIdType.MESH,
            ))

        if _VARIANT == "compute":
            for bb in range(N_B):
                kv[1, bb, 0] = k_ref[bb].astype(jnp.bfloat16)
                kv[1, bb, 1] = v_ref[bb].astype(jnp.bfloat16)

        ones = jnp.ones((S_PER, 8), jnp.bfloat16)

        qs = [
            [
                (q_ref[bb, :, hh, :] * SCALE).astype(jnp.bfloat16)
                for hh in range(N_H)
            ]
            for bb in range(N_B)
        ]

        def head_attn(slot, bb, hh):
            k = kv[slot, bb, 0, :, hh * D:(hh + 1) * D]
            s = lax.dot_general(
                qs[bb][hh], k, (((1,), (1,)), ((), ())),
                preferred_element_type=jnp.float32,
            )
            p = jnp.exp(s.astype(jnp.bfloat16))
            v = kv[slot, bb, 1, :, hh * D:(hh + 1) * D]
            acc = lax.dot_general(
                p, v, (((1,), (0,)), ((), ())),
                preferred_element_type=jnp.float32,
            )
            den = lax.dot_general(
                p, ones, (((1,), (0,)), ((), ())),
                preferred_element_type=jnp.float32,
            )
            return acc, den

        bh = [(bb, hh) for bb in range(N_B) for hh in range(N_H)]
        acc_l = {}
        unit = 0
        for c in range(N_CHUNKS):
            for _ in range(len(bh) // N_CHUNKS):
                bb, hh = bh[unit]
                acc_l[(bb, hh)] = head_attn(0, bb, hh)
                unit += 1
            if _VARIANT != "compute":
                direct[c].wait_recv()
                fwd[c].start()

        if _VARIANT == "comm":
            for c in range(N_CHUNKS):
                direct[c].wait_send()
                fwd[c].wait()
            out_ref[...] = q_ref[...]
            return

        def finish_batch(bb, parts):
            for hh in range(N_H):
                acc0, den0 = acc_l[(bb, hh)]
                acc1, den1 = parts[hh]
                acc = acc0 + acc1
                den = (den0 + den1)[:, 0:1]
                out_ref[bb, :, hh, :] = acc / den

        if _VARIANT == "compute":
            for bb in range(N_B):
                finish_batch(bb, [head_attn(1, bb, hh) for hh in range(N_H)])
            return

        b_direct = my_y
        b_fwd = 1 - my_y
        for yv in range(2):
            @pl.when(my_y == yv)
            def _():
                parts = [head_attn(1, yv, hh) for hh in range(N_H)]
                for c in range(N_CHUNKS):
                    fwd[c].wait_recv()
                finish_batch(yv, parts)
                parts2 = [head_attn(1, 1 - yv, hh) for hh in range(N_H)]
                finish_batch(1 - yv, parts2)

        for c in range(N_CHUNKS):
            direct[c].wait_send()
            fwd[c].wait_send()

    return pl.pallas_call(
        body,
        out_shape=jax.ShapeDtypeStruct((b, s_per, h, d), jnp.float32),
        in_specs=[pl.BlockSpec(memory_space=pltpu.VMEM)] * 3,
        out_specs=pl.BlockSpec(memory_space=pltpu.VMEM),
        scratch_shapes=[
            pltpu.VMEM((2, N_B, 2, S_PER, HD), jnp.bfloat16),
            pltpu.SemaphoreType.DMA((N_CHUNKS,)),
            pltpu.SemaphoreType.DMA((N_CHUNKS,)),
            pltpu.SemaphoreType.DMA((N_CHUNKS,)),
            pltpu.SemaphoreType.DMA((N_CHUNKS,)),
        ],
        compiler_params=pltpu.CompilerParams(
            collective_id=0,
            vmem_limit_bytes=100 * 1024 * 1024,
        ),
    )(Q, Kf, Vf)


# device time: 38591 ns/iter; 1.1482x vs baseline; 1.1482x over previous
import os

import jax
import jax.numpy as jnp
from jax import lax
from jax.experimental import pallas as pl
from jax.experimental.pallas import tpu as pltpu

_VARIANT = os.environ.get("KERNEL_VARIANT", "full")

N_B = 2
S_PER = 512
N_H = 8
D = 64
HD = N_H * D
SCALE = D ** -0.5

N_CHUNKS = 4
CS = S_PER // 2


def kernel(Q, K, V):
    b, s_per, h, d = Q.shape
    Qf = Q.reshape(b, s_per, h * d)
    Kf = K.reshape(b, s_per, h * d)
    Vf = V.reshape(b, s_per, h * d)

    def body(q_ref, k_ref, v_ref, out_ref, kv, dsend, drecv, fsend, frecv):
        my_x = lax.axis_index("x")
        my_y = lax.axis_index("y")
        x_partner = (1 - my_x, my_y)
        y_partner = (my_x, 1 - my_y)

        barrier = pltpu.get_barrier_semaphore()
        for nbr in (x_partner, y_partner):
            pl.semaphore_signal(
                barrier, inc=1, device_id=nbr,
                device_id_type=pl.DeviceIdType.MESH,
            )
        pl.semaphore_wait(barrier, 2)

        for bb in range(N_B):
            kv[0, bb, 0] = k_ref[bb].astype(jnp.bfloat16)
            kv[0, bb, 1] = v_ref[bb].astype(jnp.bfloat16)

        def chunk_idx(c):
            return c // 2, (c % 2) * CS

        direct = []
        for c in range(N_CHUNKS):
            t, s0 = chunk_idx(c)
            direct.append(pltpu.make_async_remote_copy(
                src_ref=kv.at[0, my_y, t, pl.ds(s0, CS)],
                dst_ref=kv.at[1, my_y, t, pl.ds(s0, CS)],
                send_sem=dsend.at[c], recv_sem=drecv.at[c],
                device_id=x_partner, device_id_type=pl.DeviceIdType.MESH,
            ))
        if _VARIANT != "compute":
            for r in direct:
                r.start()

        fwd = []
        for c in range(N_CHUNKS):
            t, s0 = chunk_idx(c)
            fwd.append(pltpu.make_async_remote_copy(
                src_ref=kv.at[1, my_y, t, pl.ds(s0, CS)],
                dst_ref=kv.at[1, my_y, t, pl.ds(s0, CS)],
                send_sem=fsend.at[c], recv_sem=frecv.at[c],
                device_id=y_partner, device_id_type=pl.DeviceIdType.MESH,
            ))

        if _VARIANT == "compute":
            for bb in range(N_B):
                kv[1, bb, 0] = k_ref[bb].astype(jnp.bfloat16)
                kv[1, bb, 1] = v_ref[bb].astype(jnp.bfloat16)

        ones = jnp.ones((S_PER, 8), jnp.bfloat16)

        qs = [
            [
                (q_ref[bb, :, hh * D:(hh + 1) * D] * SCALE).astype(jnp.bfloat16)
                for hh in range(N_H)
            ]
            for bb in range(N_B)
        ]

        def head_attn(slot, bb, hh):
            k = kv[slot, bb, 0, :, hh * D:(hh + 1) * D]
            s = lax.dot_general(
                qs[bb][hh], k, (((1,), (1,)), ((), ())),
                preferred_element_type=jnp.float32,
            )
            p = jnp.exp(s.astype(jnp.bfloat16))
            v = kv[slot, bb, 1, :, hh * D:(hh + 1) * D]
            acc = lax.dot_general(
                p, v, (((1,), (0,)), ((), ())),
                preferred_element_type=jnp.float32,
            )
            den = lax.dot_general(
                p, ones, (((1,), (0,)), ((), ())),
                preferred_element_type=jnp.float32,
            )
            return acc, den

        bh = [(bb, hh) for bb in range(N_B) for hh in range(N_H)]
        acc_l = {}
        unit = 0
        for c in range(N_CHUNKS):
            for _ in range(len(bh) // N_CHUNKS):
                bb, hh = bh[unit]
                acc_l[(bb, hh)] = head_attn(0, bb, hh)
                unit += 1
            if _VARIANT != "compute":
                direct[c].wait_recv()
                fwd[c].start()

        if _VARIANT == "comm":
            for c in range(N_CHUNKS):
                direct[c].wait_send()
                fwd[c].wait()
            out_ref[...] = q_ref[...]
            return

        def finish_batch(bb, parts):
            for hh in range(N_H):
                acc0, den0 = acc_l[(bb, hh)]
                acc1, den1 = parts[hh]
                acc = acc0 + acc1
                den = (den0 + den1)[:, 0:1]
                out_ref[bb, :, hh * D:(hh + 1) * D] = acc / den

        if _VARIANT == "compute":
            for bb in range(N_B):
                finish_batch(bb, [head_attn(1, bb, hh) for hh in range(N_H)])
            return

        b_direct = my_y
        b_fwd = 1 - my_y
        for yv in range(2):
            @pl.when(my_y == yv)
            def _():
                parts = [head_attn(1, yv, hh) for hh in range(N_H)]
                for c in range(N_CHUNKS):
                    fwd[c].wait_recv()
                finish_batch(yv, parts)
                parts2 = [head_attn(1, 1 - yv, hh) for hh in range(N_H)]
                finish_batch(1 - yv, parts2)

        for c in range(N_CHUNKS):
            direct[c].wait_send()
            fwd[c].wait_send()

    out_flat = pl.pallas_call(
        body,
        out_shape=jax.ShapeDtypeStruct((b, s_per, h * d), jnp.float32),
        in_specs=[pl.BlockSpec(memory_space=pltpu.VMEM)] * 3,
        out_specs=pl.BlockSpec(memory_space=pltpu.VMEM),
        scratch_shapes=[
            pltpu.VMEM((2, N_B, 2, S_PER, HD), jnp.bfloat16),
            pltpu.SemaphoreType.DMA((N_CHUNKS,)),
            pltpu.SemaphoreType.DMA((N_CHUNKS,)),
            pltpu.SemaphoreType.DMA((N_CHUNKS,)),
            pltpu.SemaphoreType.DMA((N_CHUNKS,)),
        ],
        compiler_params=pltpu.CompilerParams(
            collective_id=0,
            vmem_limit_bytes=100 * 1024 * 1024,
        ),
    )(Qf, Kf, Vf)
    return out_flat.reshape(b, s_per, h, d)
